# Initial kernel scaffold; baseline (speedup 1.0000x reference)
import jax
import jax.numpy as jnp
from jax.experimental import pallas as pl


def kernel(input, edge_index, edge_weight, blocks, bias):
    raise NotImplementedError("write your pallas kernel here")


# speedup vs baseline: 2.9558x; 2.9558x over previous
import functools

import jax
import jax.numpy as jnp
from jax import lax
from jax.experimental import pallas as pl
from jax.experimental.pallas import tpu as pltpu
from jax.experimental.pallas import tpu_sc as plsc

NN = 10000
D = 128
H = 64
E = 320000
GAMMA = 0.1

NC = 2
NS = 16
CH = 128
SBC = 16
NSB = 10
CH_PER_TEC = SBC * NSB
EP = NS * CH_PER_TEC * CH
ROWS_PER_TEC = NN // NS


def _tc_body(x_ref, blk_ref, y_ref):
    w = 0.9 * (1e-3 * blk_ref[0] + 1e-2 * blk_ref[1]
               + 1e-1 * blk_ref[2] + blk_ref[3])
    x = x_ref[...]
    y_ref[...] = jnp.dot(x, w, preferred_element_type=jnp.float32) + x + 1e-4


def _dense_x(inp, blocks):
    return pl.pallas_call(
        _tc_body,
        grid=(10,),
        in_specs=[
            pl.BlockSpec((1000, D), lambda i: (i, 0)),
            pl.BlockSpec((4, D, D), lambda i: (0, 0, 0)),
        ],
        out_specs=pl.BlockSpec((1000, D), lambda i: (i, 0)),
        out_shape=jax.ShapeDtypeStruct((NN, D), jnp.float32),
    )(inp, blocks)


def _sc_spmm_body(xcat, src_hbm, dst_hbm, w_hbm, bias_hbm, out_hbm,
                  src_sb, dst_sb, w_sb, rows, fill, bias_v, acc, sem):
    c = lax.axis_index("c")
    s = lax.axis_index("s")
    cvec = lax.broadcast(c, (16,))

    pltpu.sync_copy(bias_hbm.at[pl.ds(c * H, H)], bias_v)
    bvs = [bias_v[pl.ds(k * 16, 16)] for k in range(4)]

    def fill_row(i, _):
        for k in range(4):
            fill[i, pl.ds(k * 16, 16)] = bvs[k]
        return 0

    lax.fori_loop(0, ROWS_PER_TEC, fill_row, 0)
    pltpu.sync_copy(fill, acc.at[pl.ds(s * ROWS_PER_TEC, ROWS_PER_TEC)])
    plsc.subcore_barrier()

    def superchunk(sb, _):
        cb = s * CH_PER_TEC + sb * SBC
        pltpu.sync_copy(src_hbm.at[pl.ds(cb, SBC)], src_sb)
        pltpu.sync_copy(dst_hbm.at[pl.ds(cb, SBC)], dst_sb)
        pltpu.sync_copy(w_hbm.at[pl.ds(cb, SBC)], w_sb)

        def addc(a, _):
            for k in range(8):
                sl = pl.ds(k * 16, 16)
                src_sb[a, sl] = src_sb[a, sl] + cvec
            return 0

        lax.fori_loop(0, SBC, addc, 0)

        for j in range(SBC):
            pltpu.async_copy(xcat.at[src_sb.at[j]], rows, sem).wait()
            jv = jnp.full((16,), j, jnp.int32)

            def edge(e, _):
                wv = plsc.load_gather(w_sb, [jv, jnp.full((16,), e, jnp.int32)])
                for k in range(4):
                    sl = pl.ds(k * 16, 16)
                    rows[e, sl] = rows[e, sl] * wv
                return 0

            lax.fori_loop(0, CH, edge, 0)
            pltpu.sync_copy(rows, acc.at[dst_sb.at[j]], add=True)
        return 0

    lax.fori_loop(0, NSB, superchunk, 0)
    plsc.subcore_barrier()

    r0 = s * ROWS_PER_TEC
    pltpu.sync_copy(acc.at[pl.ds(r0, ROWS_PER_TEC)],
                    out_hbm.at[c, pl.ds(r0, ROWS_PER_TEC), :])


_sc_spmm = functools.partial(
    pl.kernel,
    out_type=jax.ShapeDtypeStruct((NC, NN, H), jnp.float32),
    mesh=plsc.VectorSubcoreMesh(core_axis_name="c", subcore_axis_name="s"),
    compiler_params=pltpu.CompilerParams(use_tc_tiling_on_sc=False,
                                         needs_layout_passes=False),
    scratch_types=[
        pltpu.VMEM((SBC, CH), jnp.int32),
        pltpu.VMEM((SBC, CH), jnp.int32),
        pltpu.VMEM((SBC, CH), jnp.float32),
        pltpu.VMEM((CH, H), jnp.float32),
        pltpu.VMEM((ROWS_PER_TEC, H), jnp.float32),
        pltpu.VMEM((H,), jnp.float32),
        pltpu.VMEM_SHARED((NN, H), jnp.float32),
        pltpu.SemaphoreType.DMA,
    ],
)(_sc_spmm_body)


@jax.jit
def kernel(input, edge_index, edge_weight, blocks, bias):
    y = _dense_x(input, blocks)
    xcat = y.reshape(2 * NN, H)

    pad = EP - E
    src2 = jnp.pad(edge_index[1] * 2, (0, pad)).reshape(EP // CH, CH)
    dst = jnp.pad(edge_index[0], (0, pad)).reshape(EP // CH, CH)
    w = jnp.pad(edge_weight, (0, pad)).reshape(EP // CH, CH)

    o = _sc_spmm(xcat, src2, dst, w, bias)
    return o.transpose(1, 0, 2).reshape(NN, D)
